# baseline (speedup 1.0000x reference)
import functools

import jax
import jax.numpy as jnp
from jax import lax
from jax.experimental import pallas as pl
from jax.experimental.pallas import tpu as pltpu
from jax.experimental.pallas import tpu_sc as plsc

B = 4096
D = 64
NNEG = 50
TEMP = 0.1
NC = 2
NS = 16
NW = NC * NS
BPW = B // NW
L = 16
NEG_GROUPS = (NNEG + L - 1) // L


@functools.partial(
    pl.kernel,
    out_type=(
        jax.ShapeDtypeStruct((NW * BPW // L, L), jnp.float32),
        jax.ShapeDtypeStruct((NW * BPW // L, L), jnp.float32),
    ),
    mesh=plsc.VectorSubcoreMesh(core_axis_name="c", subcore_axis_name="s"),
    compiler_params=pltpu.CompilerParams(use_tc_tiling_on_sc=False),
    scratch_types=[
        pltpu.VMEM((BPW,), jnp.int32),
        pltpu.VMEM((BPW,), jnp.int32),
        pltpu.VMEM((BPW, NNEG), jnp.int32),
        pltpu.VMEM((BPW, D), jnp.float32),
        pltpu.VMEM((BPW, D), jnp.float32),
        pltpu.VMEM((NNEG, D), jnp.float32),
        pltpu.VMEM((NNEG, D), jnp.float32),
        pltpu.VMEM((BPW // L, L), jnp.float32),
        pltpu.VMEM((BPW // L, L), jnp.float32),
        pltpu.SemaphoreType.DMA,
        pltpu.SemaphoreType.DMA,
        pltpu.SemaphoreType.DMA,
    ],
)
def _sc_scores(users_hbm, pos_hbm, neg_hbm, uemb_hbm, iemb_hbm,
               pd_out, ns_out,
               uidx, pidx, nidx, urows, prows, nb0, nb1, pd_v, ns_v,
               sem0, sem1, sem2):
    wid = lax.axis_index("s") * NC + lax.axis_index("c")
    base = wid * BPW
    lane = lax.iota(jnp.int32, L)

    pltpu.sync_copy(users_hbm.at[pl.ds(base, BPW)], uidx)
    pltpu.sync_copy(pos_hbm.at[pl.ds(base, BPW)], pidx)
    pltpu.sync_copy(neg_hbm.at[pl.ds(base, BPW)], nidx)

    cu = pltpu.make_async_copy(uemb_hbm.at[uidx], urows, sem2)
    cu.start()
    cp = pltpu.make_async_copy(iemb_hbm.at[pidx], prows, sem2)
    cp.start()
    pltpu.make_async_copy(iemb_hbm.at[nidx.at[0]], nb0, sem0).start()
    pltpu.make_async_copy(iemb_hbm.at[nidx.at[1]], nb1, sem1).start()
    cu.wait()
    cp.wait()

    for r in range(BPW // L):
        pd_v[r] = jnp.zeros((L,), jnp.float32)
        ns_v[r] = jnp.zeros((L,), jnp.float32)

    perms = {w: lane ^ w for w in (8, 4, 2, 1)}
    masks = {w: (lane & w) == 0 for w in (8, 4, 2, 1)}

    gdn = lax.GatherDimensionNumbers(
        offset_dims=(), collapsed_slice_dims=(0,), start_index_map=(0,))

    def _take(v, w):
        return lax.gather(v, perms[w][:, None], dimension_numbers=gdn,
                          slice_sizes=(1,),
                          mode=lax.GatherScatterMode.PROMISE_IN_BOUNDS)

    def _hsum(v):
        for w in (8, 4, 2, 1):
            v = v + _take(v, w)
        return v

    def _butterfly(vecs):
        for w in (8, 4, 2, 1):
            nxt = []
            for i in range(0, len(vecs), 2):
                a, c = vecs[i], vecs[i + 1]
                nxt.append(jnp.where(masks[w], a + _take(a, w), c + _take(c, w)))
            vecs = nxt
        return vecs[0]

    def do_row(b, nb, sem):
        pltpu.make_async_copy(iemb_hbm.at[nidx.at[b]], nb, sem).wait()

        u = [urows[b, pl.ds(j * L, L)] for j in range(D // L)]

        pvec = u[0] * prows[b, pl.ds(0, L)]
        for j in range(1, D // L):
            pvec = pvec + u[j] * prows[b, pl.ds(j * L, L)]
        pdv = _hsum(pvec)

        pad = jnp.full((L,), -1e29, jnp.float32)
        nacc = jnp.zeros((L,), jnp.float32)
        for g in range(NEG_GROUPS):
            cnt = min(L, NNEG - g * L)
            partials = []
            for k in range(L):
                if k < cnt:
                    n = g * L + k
                    dv = u[0] * nb[n, pl.ds(0, L)]
                    for j in range(1, D // L):
                        dv = dv + u[j] * nb[n, pl.ds(j * L, L)]
                    partials.append(dv)
                else:
                    partials.append(pad)
            sv = _butterfly(partials)
            nacc = nacc + jnp.exp(sv / jnp.float32(TEMP))
        nsv = _hsum(nacc)

        row = b // L
        sel = lane == (b % L)
        plsc.addupdate(pd_v.at[row], jnp.where(sel, pdv, jnp.float32(0.0)))
        plsc.addupdate(ns_v.at[row], jnp.where(sel, nsv, jnp.float32(0.0)))

        @pl.when(b + 2 < BPW)
        def _():
            pltpu.make_async_copy(iemb_hbm.at[nidx.at[b + 2]], nb, sem).start()

    def body(i, carry):
        b0 = i * 2
        do_row(b0, nb0, sem0)
        do_row(b0 + 1, nb1, sem1)
        return carry

    lax.fori_loop(0, BPW // 2, body, 0)

    pltpu.sync_copy(pd_v, pd_out.at[pl.ds(wid * (BPW // L), BPW // L)])
    pltpu.sync_copy(ns_v, ns_out.at[pl.ds(wid * (BPW // L), BPW // L)])


def _tc_finish_body(pd_ref, ns_ref, o_ref):
    s = pd_ref[...] / jnp.float32(TEMP)
    p = jnp.exp(s)
    loss = jnp.log(p + ns_ref[...]) - s
    o_ref[0, 0] = jnp.sum(loss) * jnp.float32(1.0 / B)


_tc_finish = pl.pallas_call(
    _tc_finish_body,
    out_shape=jax.ShapeDtypeStruct((1, 1), jnp.float32),
    out_specs=pl.BlockSpec(memory_space=pltpu.SMEM),
)


def kernel(users, positives, negatives, epoch, user_emb, item_emb):
    del epoch
    users = users.astype(jnp.int32)
    pos_flat = positives.reshape(B).astype(jnp.int32)
    negatives = negatives.astype(jnp.int32)
    pd, ns = _sc_scores(users, pos_flat, negatives, user_emb, item_emb)
    out = _tc_finish(pd.reshape(32, 128), ns.reshape(32, 128))
    return out[0, 0]
